# baseline (speedup 1.0000x reference)
import functools

import jax
import jax.numpy as jnp
from jax import lax
from jax.experimental import pallas as pl
from jax.experimental.pallas import tpu as pltpu
from jax.experimental.pallas import tpu_sc as plsc

_NC = 2
_NS = 16
_L = 16
_NW = _NC * _NS

_C = 256
_G = 128
_NB = 4


def _rsqrt(x):
    i = lax.bitcast_convert_type(x, jnp.int32)
    y = lax.bitcast_convert_type(jnp.int32(0x5F3759DF) - (i >> 1), jnp.float32)
    for _ in range(2):
        y = y * (1.5 - 0.5 * x * y * y)
    return y


def kernel(inp, W):
    B, H = inp.shape
    V, D = W.shape
    N = B * H
    per_w = N // _NW
    n_chunks = per_w // _C
    rounds = n_chunks // _NB
    n_g = _C // _G
    n_v = D // _L

    idx2d = inp.reshape(N // _G, _G)

    mesh = plsc.VectorSubcoreMesh(
        core_axis_name="c", subcore_axis_name="s",
        num_cores=_NC, num_subcores=_NS)

    @functools.partial(
        pl.kernel,
        out_type=jax.ShapeDtypeStruct((N, D), jnp.float32),
        mesh=mesh,
        scratch_types=(
            [pltpu.VMEM((_NB, n_g, _G), jnp.int32),
             pltpu.VMEM((_NB, _C, D), jnp.float32)]
            + [pltpu.SemaphoreType.DMA] * (2 * _NB)
        ),
        compiler_params=pltpu.CompilerParams(use_tc_tiling_on_sc=False),
    )
    def _emb(idx_hbm, table_hbm, out_hbm, idx_v, rows_v, *sems):
        gsem = sems[:_NB]
        wsem = sems[_NB:]
        wid = lax.axis_index("s") * _NC + lax.axis_index("c")
        lanes = lax.iota(jnp.int32, _L)

        def fire_gather(ci, b):
            pltpu.sync_copy(
                idx_hbm.at[pl.ds(wid * (per_w // _G) + ci * n_g, n_g)],
                idx_v.at[b])
            for j in range(n_g):
                pltpu.async_copy(table_hbm.at[idx_v.at[b, j]],
                                 rows_v.at[b, pl.ds(j * _G, _G)], gsem[b])

        def drain_gather(b):
            pltpu.make_async_copy(out_hbm.at[pl.ds(0, _C)], rows_v.at[b],
                                  gsem[b]).wait()

        def fire_wb(ci, b):
            pltpu.async_copy(rows_v.at[b],
                             out_hbm.at[pl.ds(wid * per_w + ci * _C, _C)],
                             wsem[b])

        def drain_wb(b):
            pltpu.make_async_copy(rows_v.at[b], out_hbm.at[pl.ds(0, _C)],
                                  wsem[b]).wait()

        def compute(b):
            @plsc.parallel_loop(0, _C, unroll=16)
            def _(r):
                v = [rows_v[b, r, pl.ds(j * _L, _L)] for j in range(n_v)]
                ss = v[0] * v[0]
                for vv in v[1:]:
                    ss = ss + vv * vv
                for k in (8, 4, 2, 1):
                    ss = ss + ss.at[lanes ^ k].get(mode="promise_in_bounds")
                rs = _rsqrt(jnp.maximum(ss, 1e-24))
                for j in range(n_v):
                    rows_v[b, r, pl.ds(j * _L, _L)] = v[j] * rs

        fire_gather(0, 0)
        fire_gather(1, 1)

        def round_body(r, _):
            for b in range(_NB):
                ci = _NB * r + b
                drain_gather(b)
                compute(b)
                fire_wb(ci, b)
                jb = (b + 2) % _NB
                cj = ci + 2
                if b < 2:
                    @pl.when(r > 0)
                    def _():
                        drain_wb(jb)
                    fire_gather(cj, jb)
                else:
                    @pl.when(r < rounds - 1)
                    def _():
                        drain_wb(jb)
                        fire_gather(cj, jb)
            return 0

        lax.fori_loop(0, rounds, round_body, 0)
        for b in range(_NB):
            drain_wb(b)

    out = _emb(idx2d, W)
    return out.reshape(B, H, D)


# speedup vs baseline: 1.0505x; 1.0505x over previous
import functools

import jax
import jax.numpy as jnp
from jax import lax
from jax.experimental import pallas as pl
from jax.experimental.pallas import tpu as pltpu
from jax.experimental.pallas import tpu_sc as plsc

_NC = 2
_NS = 16
_L = 16
_NW = _NC * _NS

_C = 256
_G = 128
_NB = 4


def _rsqrt(x):
    i = lax.bitcast_convert_type(x, jnp.int32)
    y = lax.bitcast_convert_type(jnp.int32(0x5F3759DF) - (i >> 1), jnp.float32)
    for _ in range(2):
        y = y * (1.5 - 0.5 * x * y * y)
    return y


def kernel(inp, W):
    B, H = inp.shape
    V, D = W.shape
    N = B * H
    per_w = N // _NW
    n_chunks = per_w // _C
    rounds = n_chunks // _NB
    n_g = _C // _G
    n_v = D // _L

    idx2d = inp.reshape(N // _G, _G)

    mesh = plsc.VectorSubcoreMesh(
        core_axis_name="c", subcore_axis_name="s",
        num_cores=_NC, num_subcores=_NS)

    @functools.partial(
        pl.kernel,
        out_type=jax.ShapeDtypeStruct((N, D), jnp.float32),
        mesh=mesh,
        scratch_types=(
            [pltpu.VMEM((_NB, n_g, _G), jnp.int32),
             pltpu.VMEM((_NB, _C, D), jnp.float32)]
            + [pltpu.SemaphoreType.DMA] * (2 * _NB)
        ),
        compiler_params=pltpu.CompilerParams(use_tc_tiling_on_sc=False),
    )
    def _emb(idx_hbm, table_hbm, out_hbm, idx_v, rows_v, *sems):
        gsem = sems[:_NB]
        wsem = sems[_NB:]
        wid = lax.axis_index("s") * _NC + lax.axis_index("c")
        lanes = lax.iota(jnp.int32, _L)

        def fire_gather(ci, b):
            pltpu.sync_copy(
                idx_hbm.at[pl.ds(wid * (per_w // _G) + ci * n_g, n_g)],
                idx_v.at[b])
            for j in range(n_g):
                pltpu.async_copy(table_hbm.at[idx_v.at[b, j]],
                                 rows_v.at[b, pl.ds(j * _G, _G)], gsem[b])

        def drain_gather(b):
            pltpu.make_async_copy(out_hbm.at[pl.ds(0, _C)], rows_v.at[b],
                                  gsem[b]).wait()

        def fire_wb(ci, b):
            pltpu.async_copy(rows_v.at[b],
                             out_hbm.at[pl.ds(wid * per_w + ci * _C, _C)],
                             wsem[b])

        def drain_wb(b):
            pltpu.make_async_copy(rows_v.at[b], out_hbm.at[pl.ds(0, _C)],
                                  wsem[b]).wait()

        def compute(b):
            @plsc.parallel_loop(0, _C, unroll=8)
            def _(r):
                v = [rows_v[b, r, pl.ds(j * _L, _L)] for j in range(n_v)]
                ss = v[0] * v[0]
                for vv in v[1:]:
                    ss = ss + vv * vv
                for k in (8, 4, 2, 1):
                    ss = ss + ss.at[lanes ^ k].get(mode="promise_in_bounds")
                rs = _rsqrt(jnp.maximum(ss, 1e-24))
                for j in range(n_v):
                    rows_v[b, r, pl.ds(j * _L, _L)] = v[j] * rs

        fire_gather(0, 0)
        fire_gather(1, 1)

        def round_body(r, _):
            for b in range(_NB):
                ci = _NB * r + b
                drain_gather(b)
                compute(b)
                fire_wb(ci, b)
                jb = (b + 2) % _NB
                cj = ci + 2
                if b < 2:
                    @pl.when(r > 0)
                    def _():
                        drain_wb(jb)
                    fire_gather(cj, jb)
                else:
                    @pl.when(r < rounds - 1)
                    def _():
                        drain_wb(jb)
                        fire_gather(cj, jb)
            return 0

        lax.fori_loop(0, rounds, round_body, 0)
        for b in range(_NB):
            drain_wb(b)

    out = _emb(idx2d, W)
    return out.reshape(B, H, D)
